# baseline (speedup 1.0000x reference)
import functools

import jax
import jax.numpy as jnp
from jax import lax
from jax.experimental import pallas as pl
from jax.experimental.pallas import tpu as pltpu
from jax.experimental.pallas import tpu_sc as plsc

N_NODES = 10000
N_EDGES = 320000
D = 128

NC = 2
NS = 16
NW = NC * NS

EDGES_PER_TILE = N_EDGES // NW
CHUNK = 112
CHUNKS = 90
PAD_EDGES = CHUNKS * CHUNK - EDGES_PER_TILE
N_PAD = 10112
ROWS_PER_TILE = N_PAD // NS


def _sc_aggregate(feature, pk3):
    mesh = plsc.VectorSubcoreMesh(core_axis_name="c", subcore_axis_name="s")

    @functools.partial(
        pl.kernel,
        mesh=mesh,
        out_type=jax.ShapeDtypeStruct((NC, N_PAD, D), jnp.float32),
        scratch_types=[
            pltpu.VMEM((CHUNKS, CHUNK), jnp.int32),
            pltpu.VMEM((1, CHUNK), jnp.int32),
            pltpu.VMEM((1, CHUNK), jnp.int32),
            pltpu.VMEM((1, CHUNK), jnp.int32),
            pltpu.VMEM((1, CHUNK), jnp.int32),
            pltpu.VMEM((CHUNK, D), jnp.float32),
            pltpu.VMEM((CHUNK, D), jnp.float32),
            pltpu.VMEM_SHARED((N_PAD, D), jnp.float32),
            pltpu.SemaphoreType.DMA,
        ],
    )
    def agg(feat_hbm, pk_hbm, out_hbm, pk_v, s0, s1, d0, d1, rows0, rows1,
            acc_sh, gsem):
        c = lax.axis_index("c")
        s = lax.axis_index("s")
        wid = c * NS + s

        for r in range(CHUNK):
            for k in range(D // 16):
                rows0[r, pl.ds(k * 16, 16)] = jnp.zeros((16,), jnp.float32)
        for j in range(ROWS_PER_TILE // CHUNK):
            pltpu.sync_copy(rows0,
                            acc_sh.at[pl.ds(s * ROWS_PER_TILE + j * CHUNK, CHUNK)])
        rem = ROWS_PER_TILE % CHUNK
        if rem:
            pltpu.sync_copy(
                rows0.at[pl.ds(0, rem)],
                acc_sh.at[pl.ds(s * ROWS_PER_TILE + ROWS_PER_TILE - rem, rem)])

        pltpu.sync_copy(pk_hbm.at[wid], pk_v)
        plsc.subcore_barrier()

        def unpack(ci, sbuf, dbuf):
            for k in range(CHUNK // 16):
                x = pk_v[ci, pl.ds(k * 16, 16)]
                sbuf[0, pl.ds(k * 16, 16)] = jnp.bitwise_and(
                    x, jnp.int32(0xFFFF))
                dbuf[0, pl.ds(k * 16, 16)] = lax.shift_right_logical(
                    x, jnp.int32(16))

        def gcopy(sbuf, buf):
            return pltpu.make_async_copy(feat_hbm.at[sbuf.at[0]], buf, gsem)

        unpack(0, s0, d0)
        gcopy(s0, rows0).start()

        def pair_body(p, carry):
            c0 = 2 * p
            unpack(c0 + 1, s1, d1)
            gcopy(s0, rows0).wait()
            gcopy(s1, rows1).start()
            pltpu.sync_copy(rows0, acc_sh.at[d0.at[0]], add=True)

            @pl.when(p + 1 < CHUNKS // 2)
            def _():
                unpack(c0 + 2, s0, d0)
                gcopy(s1, rows1).wait()
                gcopy(s0, rows0).start()

            @pl.when(p + 1 >= CHUNKS // 2)
            def _():
                gcopy(s1, rows1).wait()

            pltpu.sync_copy(rows1, acc_sh.at[d1.at[0]], add=True)
            return carry

        lax.fori_loop(0, CHUNKS // 2, pair_body, 0)
        plsc.subcore_barrier()

        pltpu.sync_copy(acc_sh.at[pl.ds(s * ROWS_PER_TILE, ROWS_PER_TILE)],
                        out_hbm.at[c, pl.ds(s * ROWS_PER_TILE, ROWS_PER_TILE)])

    return agg(feature, pk3)


def _linear_body(h2_ref, w_ref, b_ref, o_ref):
    h = h2_ref[0] + h2_ref[1]
    o_ref[...] = lax.dot_general(
        h, w_ref[...], (((1,), (1,)), ((), ())),
        preferred_element_type=jnp.float32) + b_ref[...]


def _linear(partials, W, b2):
    blk = 2000
    return pl.pallas_call(
        _linear_body,
        grid=(N_NODES // blk,),
        in_specs=[
            pl.BlockSpec((NC, blk, D), lambda i: (0, i, 0)),
            pl.BlockSpec((D, D), lambda i: (0, 0)),
            pl.BlockSpec((1, D), lambda i: (0, 0)),
        ],
        out_specs=pl.BlockSpec((blk, D), lambda i: (i, 0)),
        out_shape=jax.ShapeDtypeStruct((N_NODES, D), jnp.float32),
    )(partials, W, b2)


def kernel(feature, edge_index, W, b):
    src = edge_index[0].astype(jnp.int32).reshape(NW, EDGES_PER_TILE)
    dst = edge_index[1].astype(jnp.int32).reshape(NW, EDGES_PER_TILE)
    pk = jnp.bitwise_or(src, jnp.left_shift(dst, 16))
    pad_dst = N_NODES + (jnp.arange(PAD_EDGES, dtype=jnp.int32)
                         % (N_PAD - N_NODES))
    pad_pk = jnp.broadcast_to(jnp.left_shift(pad_dst, 16), (NW, PAD_EDGES))
    pk3 = jnp.concatenate([pk, pad_pk], axis=1).reshape(NW, CHUNKS, CHUNK)
    partials = _sc_aggregate(feature, pk3)
    return _linear(partials, W, b.reshape(1, D))


# speedup vs baseline: 1.2160x; 1.2160x over previous
import functools

import jax
import jax.numpy as jnp
from jax import lax
from jax.experimental import pallas as pl
from jax.experimental.pallas import tpu as pltpu
from jax.experimental.pallas import tpu_sc as plsc

N_NODES = 10000
N_EDGES = 320000
D = 128

NC = 2
NS = 16
NW = NC * NS

EDGES_PER_TILE = N_EDGES // NW
CHUNK = 125
CHUNKS = EDGES_PER_TILE // CHUNK
N_PAD = 10112
ROWS_PER_TILE = N_PAD // NS


def _sc_aggregate(feature, src3, dst3):
    mesh = plsc.VectorSubcoreMesh(core_axis_name="c", subcore_axis_name="s")

    @functools.partial(
        pl.kernel,
        mesh=mesh,
        out_type=jax.ShapeDtypeStruct((NC, N_PAD, D), jnp.float32),
        scratch_types=[
            pltpu.VMEM((CHUNKS, CHUNK), jnp.int32),
            pltpu.VMEM((CHUNKS, CHUNK), jnp.int32),
            pltpu.VMEM((CHUNK, D), jnp.float32),
            pltpu.VMEM_SHARED((N_PAD, D), jnp.float32),
            pltpu.SemaphoreType.DMA,
        ],
    )
    def agg(feat_hbm, src_hbm, dst_hbm, out_hbm, src_v, dst_v, rows_v,
            acc_sh, gsem):
        c = lax.axis_index("c")
        s = lax.axis_index("s")
        wid = c * NS + s

        for r in range(CHUNK):
            for k in range(D // 16):
                rows_v[r, pl.ds(k * 16, 16)] = jnp.zeros((16,), jnp.float32)
        for j in range(ROWS_PER_TILE // CHUNK):
            pltpu.sync_copy(rows_v,
                            acc_sh.at[pl.ds(s * ROWS_PER_TILE + j * CHUNK, CHUNK)])
        rem = ROWS_PER_TILE % CHUNK
        if rem:
            pltpu.sync_copy(
                rows_v.at[pl.ds(0, rem)],
                acc_sh.at[pl.ds(s * ROWS_PER_TILE + ROWS_PER_TILE - rem, rem)])

        pltpu.sync_copy(src_hbm.at[wid], src_v)
        pltpu.sync_copy(dst_hbm.at[wid], dst_v)
        plsc.subcore_barrier()

        def chunk_body(ci, carry):
            ci2 = lax.rem(ci + s * 5, CHUNKS)
            pltpu.async_copy(feat_hbm.at[src_v.at[ci2]], rows_v, gsem).wait()
            pltpu.sync_copy(rows_v, acc_sh.at[dst_v.at[ci2]], add=True)
            return carry

        lax.fori_loop(0, CHUNKS, chunk_body, 0)
        plsc.subcore_barrier()

        pltpu.sync_copy(acc_sh.at[pl.ds(s * ROWS_PER_TILE, ROWS_PER_TILE)],
                        out_hbm.at[c, pl.ds(s * ROWS_PER_TILE, ROWS_PER_TILE)])

    return agg(feature, src3, dst3)


def _linear_body(h2_ref, w_ref, b_ref, o_ref):
    h = h2_ref[0] + h2_ref[1]
    o_ref[...] = lax.dot_general(
        h, w_ref[...], (((1,), (1,)), ((), ())),
        preferred_element_type=jnp.float32) + b_ref[...]


def _linear(partials, W, b2):
    blk = 2000
    return pl.pallas_call(
        _linear_body,
        grid=(N_NODES // blk,),
        in_specs=[
            pl.BlockSpec((NC, blk, D), lambda i: (0, i, 0)),
            pl.BlockSpec((D, D), lambda i: (0, 0)),
            pl.BlockSpec((1, D), lambda i: (0, 0)),
        ],
        out_specs=pl.BlockSpec((blk, D), lambda i: (i, 0)),
        out_shape=jax.ShapeDtypeStruct((N_NODES, D), jnp.float32),
    )(partials, W, b2)


def kernel(feature, edge_index, W, b):
    src3 = edge_index[0].astype(jnp.int32).reshape(NW, CHUNKS, CHUNK)
    dst3 = edge_index[1].astype(jnp.int32).reshape(NW, CHUNKS, CHUNK)
    partials = _sc_aggregate(feature, src3, dst3)
    return _linear(partials, W, b.reshape(1, D))
